# Initial kernel scaffold; baseline (speedup 1.0000x reference)
import jax
import jax.numpy as jnp
from jax.experimental import pallas as pl


def kernel(nfeat, efeat, n2p_idx, n2p_val, e2p_idx, e2p_val, pool_idx, pool_val, degs, weights, bias, W0, b0, W1, b1):
    raise NotImplementedError("write your pallas kernel here")


# speedup vs baseline: 20.3147x; 20.3147x over previous
import functools

import jax
import jax.numpy as jnp
from jax import lax
from jax.experimental import pallas as pl
from jax.experimental.pallas import tpu as pltpu
from jax.experimental.pallas import tpu_sc as plsc

N = 10000
E = 320000
P = 100000
L = 16
IN = 16
OUT = 256

NC = 2
NS = 16
NW = NC * NS

P2 = 100352
M2 = P2 * L
CHUNK = 1024
CHUNKS_PER_W = M2 // (NW * CHUNK)
ROWS_PER_W = CHUNK * CHUNKS_PER_W


def _gather_body(nfeat_hbm, efeat_hbm, idxn_hbm, idxe_hbm, valn_hbm, vale_hbm,
                 y_hbm, idxn_v, idxe_v, valn_v, vale_v, gn_v, ge_v, y_v, sem):
    wid = lax.axis_index("s") * NC + lax.axis_index("c")
    base0 = wid * ROWS_PER_W

    def chunk_body(k, _):
        base = pl.multiple_of(base0 + k * CHUNK, CHUNK)
        brow = pl.multiple_of(base // 128, 8)
        pltpu.sync_copy(idxn_hbm.at[pl.ds(brow, 8), :], idxn_v)
        pltpu.sync_copy(idxe_hbm.at[pl.ds(brow, 8), :], idxe_v)
        pltpu.sync_copy(valn_hbm.at[pl.ds(base, CHUNK)], valn_v)
        pltpu.sync_copy(vale_hbm.at[pl.ds(base, CHUNK)], vale_v)
        copies = []
        for j in range(8):
            copies.append(pltpu.async_copy(
                nfeat_hbm.at[idxn_v.at[j]], gn_v.at[pl.ds(j * 128, 128)], sem))
            copies.append(pltpu.async_copy(
                efeat_hbm.at[idxe_v.at[j]], ge_v.at[pl.ds(j * 128, 128)], sem))
        for c in copies:
            c.wait()

        def row_body(t, _):
            va = valn_v[pl.ds(t * 16, 16)]
            vb = vale_v[pl.ds(t * 16, 16)]
            for u in range(16):
                r = t * 16 + u
                y_v[r, :] = va[u] * gn_v[r, :] + vb[u] * ge_v[r, :]
            return 0

        lax.fori_loop(0, CHUNK // 16, row_body, 0)
        pltpu.sync_copy(y_v, y_hbm.at[pl.ds(base, CHUNK), :])
        return 0

    lax.fori_loop(0, CHUNKS_PER_W, chunk_body, 0)


def _sc_gather(nfeat, efeat, idxn, idxe, valn, vale):
    mesh = plsc.VectorSubcoreMesh(core_axis_name="c", subcore_axis_name="s",
                                  num_cores=NC, num_subcores=NS)
    f = pl.kernel(
        _gather_body,
        out_type=jax.ShapeDtypeStruct((M2, IN), jnp.float32),
        mesh=mesh,
        compiler_params=pltpu.CompilerParams(use_tc_tiling_on_sc=False),
        scratch_types=[
            pltpu.VMEM((8, 128), jnp.int32),
            pltpu.VMEM((8, 128), jnp.int32),
            pltpu.VMEM((CHUNK,), jnp.float32),
            pltpu.VMEM((CHUNK,), jnp.float32),
            pltpu.VMEM((CHUNK, IN), jnp.float32),
            pltpu.VMEM((CHUNK, IN), jnp.float32),
            pltpu.VMEM((CHUNK, IN), jnp.float32),
            pltpu.SemaphoreType.DMA,
        ],
    )
    return f(nfeat, efeat, idxn.reshape(M2 // 128, 128),
             idxe.reshape(M2 // 128, 128), valn, vale)


TP = 2048


def _mm_body(x_ref, w_ref, b_ref, pv_ref, z_ref):
    acc = jnp.dot(x_ref[...], w_ref[...], preferred_element_type=jnp.float32)
    z_ref[...] = jnp.maximum(acc + b_ref[...], 0.0) * pv_ref[...]


def _tc_matmul(x_flat, w_flat, bias, pool_val2d):
    grid = (P2 // TP,)
    return pl.pallas_call(
        _mm_body,
        grid=grid,
        in_specs=[
            pl.BlockSpec((TP, L * IN), lambda i: (i, 0)),
            pl.BlockSpec((L * IN, OUT), lambda i: (0, 0)),
            pl.BlockSpec((1, OUT), lambda i: (0, 0)),
            pl.BlockSpec((TP, 1), lambda i: (i, 0)),
        ],
        out_specs=pl.BlockSpec((TP, OUT), lambda i: (i, 0)),
        out_shape=jax.ShapeDtypeStruct((P2, OUT), jnp.float32),
    )(x_flat, w_flat, bias, pool_val2d)


HALF = OUT // NC
SROWS = P2 // NS
SCHUNKS = SROWS // 128


ZROWS = 208


def _pool_body(z_hbm, idx_hbm, acc_out_hbm, idx_v, row_v, zero_v, acc_sh, sem):
    cid = lax.axis_index("c")
    sid = lax.axis_index("s")
    c0 = pl.multiple_of(cid * HALF, HALF)

    def zrow(r, _):
        for j in range(HALF // 16):
            zero_v[r, pl.ds(j * 16, 16)] = jnp.zeros((16,), jnp.float32)
        return 0

    lax.fori_loop(0, ZROWS, zrow, 0)
    a0 = pl.multiple_of(sid * 624, 8)
    for j in range(3):
        pltpu.sync_copy(zero_v, acc_sh.at[pl.ds(a0 + j * ZROWS, ZROWS), :])

    @pl.when(sid == NS - 1)
    def _():
        pltpu.sync_copy(zero_v.at[pl.ds(0, 16), :],
                        acc_sh.at[pl.ds(N - 16, 16), :])

    plsc.subcore_barrier()

    pltpu.sync_copy(idx_hbm.at[sid], idx_v)

    r0 = pl.multiple_of(sid * SROWS, 128)

    def chunk_body(k, _):
        zr = pl.multiple_of(r0 + k * 128, 128)
        pltpu.sync_copy(z_hbm.at[pl.ds(zr, 128), pl.ds(c0, HALF)], row_v)
        pltpu.sync_copy(row_v, acc_sh.at[idx_v.at[k]], add=True)
        return 0

    lax.fori_loop(0, SCHUNKS, chunk_body, 0)
    plsc.subcore_barrier()

    pltpu.sync_copy(acc_sh.at[pl.ds(a0, 624), :],
                    acc_out_hbm.at[pl.ds(a0, 624), pl.ds(c0, HALF)])

    @pl.when(sid == NS - 1)
    def _():
        pltpu.sync_copy(acc_sh.at[pl.ds(N - 16, 16), :],
                        acc_out_hbm.at[pl.ds(N - 16, 16), pl.ds(c0, HALF)])


def _sc_pool(z, pool_idx2d):
    mesh = plsc.VectorSubcoreMesh(core_axis_name="c", subcore_axis_name="s",
                                  num_cores=NC, num_subcores=NS)
    f = pl.kernel(
        _pool_body,
        out_type=jax.ShapeDtypeStruct((N, OUT), jnp.float32),
        mesh=mesh,
        scratch_types=[
            pltpu.VMEM((SCHUNKS, 128), jnp.int32),
            pltpu.VMEM((128, HALF), jnp.float32),
            pltpu.VMEM((ZROWS, HALF), jnp.float32),
            pltpu.VMEM_SHARED((N, HALF), jnp.float32),
            pltpu.SemaphoreType.DMA,
        ],
    )
    return f(z, pool_idx2d)


TN = 2000


def _final_body(acc_ref, degs_ref, w0_ref, b0_ref, w1_ref, b1_ref, out_ref):
    h = jnp.maximum(degs_ref[...] * w0_ref[...] + b0_ref[...], 0.0)
    factor = jnp.dot(h, w1_ref[...], preferred_element_type=jnp.float32)
    factor = factor + b1_ref[...]
    out_ref[...] = jnp.maximum(acc_ref[...] * factor, 0.0)


def _tc_final(acc, degs2d, W0, b0_2d, W1, b1_2d):
    grid = (N // TN,)
    return pl.pallas_call(
        _final_body,
        grid=grid,
        in_specs=[
            pl.BlockSpec((TN, OUT), lambda i: (i, 0)),
            pl.BlockSpec((TN, 1), lambda i: (i, 0)),
            pl.BlockSpec((1, 2 * OUT), lambda i: (0, 0)),
            pl.BlockSpec((1, 2 * OUT), lambda i: (0, 0)),
            pl.BlockSpec((2 * OUT, OUT), lambda i: (0, 0)),
            pl.BlockSpec((1, OUT), lambda i: (0, 0)),
        ],
        out_specs=pl.BlockSpec((TN, OUT), lambda i: (i, 0)),
        out_shape=jax.ShapeDtypeStruct((N, OUT), jnp.float32),
    )(acc, degs2d, W0, b0_2d, W1, b1_2d)


def kernel(nfeat, efeat, n2p_idx, n2p_val, e2p_idx, e2p_val, pool_idx,
           pool_val, degs, weights, bias, W0, b0, W1, b1):
    mpad = M2 - n2p_idx.shape[0]
    ppad = P2 - pool_idx.shape[0]
    idxn = jnp.concatenate([n2p_idx.astype(jnp.int32),
                            jnp.zeros((mpad,), jnp.int32)])
    idxe = jnp.concatenate([e2p_idx.astype(jnp.int32),
                            jnp.zeros((mpad,), jnp.int32)])
    valn = jnp.concatenate([n2p_val, jnp.zeros((mpad,), jnp.float32)])
    vale = jnp.concatenate([e2p_val, jnp.zeros((mpad,), jnp.float32)])
    pidx = jnp.concatenate([pool_idx.astype(jnp.int32),
                            jnp.zeros((ppad,), jnp.int32)])
    pval = jnp.concatenate([pool_val, jnp.zeros((ppad,), jnp.float32)])

    y = _sc_gather(nfeat, efeat, idxn, idxe, valn, vale)
    x_flat = y.reshape(P2, L * IN)
    w_flat = weights.transpose(2, 0, 1).reshape(L * IN, OUT)
    z = _tc_matmul(x_flat, w_flat, bias, pval.reshape(P2, 1))
    acc = _sc_pool(z, pidx.reshape(NS, SCHUNKS, 128))
    out = _tc_final(acc, degs.reshape(N, 1), W0, b0.reshape(1, 2 * OUT),
                    W1, b1.reshape(1, OUT))
    return out


# speedup vs baseline: 21.0594x; 1.0367x over previous
import functools

import jax
import jax.numpy as jnp
from jax import lax
from jax.experimental import pallas as pl
from jax.experimental.pallas import tpu as pltpu
from jax.experimental.pallas import tpu_sc as plsc

N = 10000
E = 320000
P = 100000
L = 16
IN = 16
OUT = 256

NC = 2
NS = 16
NW = NC * NS

P2 = 100352
M2 = P2 * L
CHUNK = 1024
CHUNKS_PER_W = M2 // (NW * CHUNK)
ROWS_PER_W = CHUNK * CHUNKS_PER_W


def _gather_body(nfeat_hbm, efeat_hbm, idxn_hbm, idxe_hbm, valn_hbm, vale_hbm,
                 y_hbm, idxn_v, idxe_v, valn_v, vale_v, gn_v, ge_v, y_v,
                 semI, semG, semW):
    wid = lax.axis_index("s") * NC + lax.axis_index("c")
    base0 = wid * ROWS_PER_W
    NCH = CHUNKS_PER_W

    def issue_inputs(k, p):
        base = pl.multiple_of(base0 + k * CHUNK, CHUNK)
        brow = pl.multiple_of(base // 128, 8)
        hs = [pltpu.async_copy(idxn_hbm.at[pl.ds(brow, 8), :],
                               idxn_v.at[p], semI),
              pltpu.async_copy(idxe_hbm.at[pl.ds(brow, 8), :],
                               idxe_v.at[p], semI),
              pltpu.async_copy(valn_hbm.at[pl.ds(base, CHUNK)],
                               valn_v.at[p], semI),
              pltpu.async_copy(vale_hbm.at[pl.ds(base, CHUNK)],
                               vale_v.at[p], semI)]
        return hs

    def issue_gathers(p):
        hs = []
        for j in range(8):
            hs.append(pltpu.async_copy(
                nfeat_hbm.at[idxn_v.at[p, j]],
                gn_v.at[p, pl.ds(j * 128, 128), :], semG))
            hs.append(pltpu.async_copy(
                efeat_hbm.at[idxe_v.at[p, j]],
                ge_v.at[p, pl.ds(j * 128, 128), :], semG))
        return hs

    def wait_inputs(p):
        for h in issue_inputs_descs(p):
            h.wait()

    def issue_inputs_descs(p):
        base = pl.multiple_of(base0, CHUNK)
        brow = pl.multiple_of(base // 128, 8)
        return [pltpu.make_async_copy(idxn_hbm.at[pl.ds(brow, 8), :],
                                      idxn_v.at[p], semI),
                pltpu.make_async_copy(idxe_hbm.at[pl.ds(brow, 8), :],
                                      idxe_v.at[p], semI),
                pltpu.make_async_copy(valn_hbm.at[pl.ds(base, CHUNK)],
                                      valn_v.at[p], semI),
                pltpu.make_async_copy(vale_hbm.at[pl.ds(base, CHUNK)],
                                      vale_v.at[p], semI)]

    def wait_gathers(p):
        for j in range(8):
            pltpu.make_async_copy(
                nfeat_hbm.at[idxn_v.at[p, j]],
                gn_v.at[p, pl.ds(j * 128, 128), :], semG).wait()
            pltpu.make_async_copy(
                efeat_hbm.at[idxe_v.at[p, j]],
                ge_v.at[p, pl.ds(j * 128, 128), :], semG).wait()

    def drain_writeback(p):
        pltpu.make_async_copy(
            y_v.at[p], y_hbm.at[pl.ds(0, CHUNK), :], semW).wait()

    for h in issue_inputs(0, 0):
        h.wait()
    issue_gathers(0)
    issue_inputs(1, 1)

    def chunk_body(k, _):
        p = lax.rem(k, 2)
        p1 = 1 - p
        base = pl.multiple_of(base0 + k * CHUNK, CHUNK)
        wait_gathers(p)

        @pl.when(k + 1 < NCH)
        def _():
            wait_inputs(p1)
            issue_gathers(p1)

        @pl.when(k >= 2)
        def _():
            drain_writeback(p)

        def row_body(t, _):
            va = valn_v[p, pl.ds(t * 16, 16)]
            vb = vale_v[p, pl.ds(t * 16, 16)]
            for u in range(16):
                r = t * 16 + u
                y_v[p, r, :] = va[u] * gn_v[p, r, :] + vb[u] * ge_v[p, r, :]
            return 0

        lax.fori_loop(0, CHUNK // 16, row_body, 0)
        pltpu.async_copy(y_v.at[p], y_hbm.at[pl.ds(base, CHUNK), :], semW)

        @pl.when(k + 2 < NCH)
        def _():
            issue_inputs(k + 2, p)

        return 0

    lax.fori_loop(0, NCH, chunk_body, 0)
    drain_writeback(0)
    drain_writeback(1)


def _sc_gather(nfeat, efeat, idxn, idxe, valn, vale):
    mesh = plsc.VectorSubcoreMesh(core_axis_name="c", subcore_axis_name="s",
                                  num_cores=NC, num_subcores=NS)
    f = pl.kernel(
        _gather_body,
        out_type=jax.ShapeDtypeStruct((M2, IN), jnp.float32),
        mesh=mesh,
        compiler_params=pltpu.CompilerParams(use_tc_tiling_on_sc=False),
        scratch_types=[
            pltpu.VMEM((2, 8, 128), jnp.int32),
            pltpu.VMEM((2, 8, 128), jnp.int32),
            pltpu.VMEM((2, CHUNK), jnp.float32),
            pltpu.VMEM((2, CHUNK), jnp.float32),
            pltpu.VMEM((2, CHUNK, IN), jnp.float32),
            pltpu.VMEM((2, CHUNK, IN), jnp.float32),
            pltpu.VMEM((2, CHUNK, IN), jnp.float32),
            pltpu.SemaphoreType.DMA,
            pltpu.SemaphoreType.DMA,
            pltpu.SemaphoreType.DMA,
        ],
    )
    return f(nfeat, efeat, idxn.reshape(M2 // 128, 128),
             idxe.reshape(M2 // 128, 128), valn, vale)


TP = 2048


def _mm_body(x_ref, w_ref, b_ref, pv_ref, z_ref):
    acc = jnp.dot(x_ref[...], w_ref[...], preferred_element_type=jnp.float32)
    z_ref[...] = jnp.maximum(acc + b_ref[...], 0.0) * pv_ref[...]


def _tc_matmul(x_flat, w_flat, bias, pool_val2d):
    grid = (P2 // TP,)
    return pl.pallas_call(
        _mm_body,
        grid=grid,
        in_specs=[
            pl.BlockSpec((TP, L * IN), lambda i: (i, 0)),
            pl.BlockSpec((L * IN, OUT), lambda i: (0, 0)),
            pl.BlockSpec((1, OUT), lambda i: (0, 0)),
            pl.BlockSpec((TP, 1), lambda i: (i, 0)),
        ],
        out_specs=pl.BlockSpec((TP, OUT), lambda i: (i, 0)),
        out_shape=jax.ShapeDtypeStruct((P2, OUT), jnp.float32),
    )(x_flat, w_flat, bias, pool_val2d)


HALF = OUT // NC
SROWS = P2 // NS
PCHUNK = 112
PCH = SROWS // PCHUNK
SUB = 112
NSUB = PCHUNK // SUB


def _pool_body(z_hbm, idx_hbm, acc_out_hbm, idx_v, row_v, acc_sh,
               semR, semS):
    cid = lax.axis_index("c")
    sid = lax.axis_index("s")
    c0 = pl.multiple_of(cid * HALF, HALF)
    r0 = pl.multiple_of(sid * SROWS, 8)

    def issue_read(k, p):
        zr = pl.multiple_of(r0 + k * PCHUNK, 8)
        return pltpu.async_copy(
            z_hbm.at[pl.ds(zr, PCHUNK), pl.ds(c0, HALF)], row_v.at[p], semR)

    def wait_read(p):
        pltpu.make_async_copy(
            z_hbm.at[pl.ds(r0, PCHUNK), pl.ds(c0, HALF)], row_v.at[p],
            semR).wait()

    def issue_scatters(k, p):
        for j in range(NSUB):
            pltpu.async_copy(
                row_v.at[p, pl.ds(j * SUB, SUB), :],
                acc_sh.at[idx_v.at[k * NSUB + j]], semS, add=True)

    def drain_scatters(p):
        for j in range(NSUB):
            pltpu.make_async_copy(
                row_v.at[p, pl.ds(j * SUB, SUB), :],
                acc_sh.at[idx_v.at[j]], semS).wait()

    def zrow(r, _):
        for j in range(HALF // 16):
            row_v[0, r, pl.ds(j * 16, 16)] = jnp.zeros((16,), jnp.float32)
        return 0

    lax.fori_loop(0, PCHUNK, zrow, 0)
    a0 = pl.multiple_of(sid * 624, 8)
    for j in range(5):
        pltpu.sync_copy(row_v.at[0],
                        acc_sh.at[pl.ds(a0 + j * PCHUNK, PCHUNK), :])
    pltpu.sync_copy(row_v.at[0, pl.ds(0, 64), :],
                    acc_sh.at[pl.ds(a0 + 5 * PCHUNK, 64), :])

    @pl.when(sid == NS - 1)
    def _():
        pltpu.sync_copy(row_v.at[0, pl.ds(0, 16), :],
                        acc_sh.at[pl.ds(N - 16, 16), :])

    pltpu.sync_copy(idx_hbm.at[sid], idx_v)
    plsc.subcore_barrier()
    issue_read(0, 0)

    def chunk_body(k, _):
        p = lax.rem(k, 2)
        p1 = 1 - p
        wait_read(p)

        @pl.when(k >= 1)
        def _():
            drain_scatters(p1)

        @pl.when(k + 1 < PCH)
        def _():
            issue_read(k + 1, p1)

        issue_scatters(k, p)
        return 0

    lax.fori_loop(0, PCH, chunk_body, 0)
    drain_scatters((PCH - 1) % 2)
    plsc.subcore_barrier()

    pltpu.sync_copy(acc_sh.at[pl.ds(a0, 624), :],
                    acc_out_hbm.at[pl.ds(a0, 624), pl.ds(c0, HALF)])

    @pl.when(sid == NS - 1)
    def _():
        pltpu.sync_copy(acc_sh.at[pl.ds(N - 16, 16), :],
                        acc_out_hbm.at[pl.ds(N - 16, 16), pl.ds(c0, HALF)])


def _sc_pool(z, pool_idx2d):
    mesh = plsc.VectorSubcoreMesh(core_axis_name="c", subcore_axis_name="s",
                                  num_cores=NC, num_subcores=NS)
    f = pl.kernel(
        _pool_body,
        out_type=jax.ShapeDtypeStruct((N, OUT), jnp.float32),
        mesh=mesh,
        scratch_types=[
            pltpu.VMEM((PCH * NSUB, SUB), jnp.int32),
            pltpu.VMEM((2, PCHUNK, HALF), jnp.float32),
            pltpu.VMEM_SHARED((N, HALF), jnp.float32),
            pltpu.SemaphoreType.DMA,
            pltpu.SemaphoreType.DMA,
        ],
    )
    return f(z, pool_idx2d)


TN = 2000


def _final_body(acc_ref, degs_ref, w0_ref, b0_ref, w1_ref, b1_ref, out_ref):
    h = jnp.maximum(degs_ref[...] * w0_ref[...] + b0_ref[...], 0.0)
    factor = jnp.dot(h, w1_ref[...], preferred_element_type=jnp.float32)
    factor = factor + b1_ref[...]
    out_ref[...] = jnp.maximum(acc_ref[...] * factor, 0.0)


def _tc_final(acc, degs2d, W0, b0_2d, W1, b1_2d):
    grid = (N // TN,)
    return pl.pallas_call(
        _final_body,
        grid=grid,
        in_specs=[
            pl.BlockSpec((TN, OUT), lambda i: (i, 0)),
            pl.BlockSpec((TN, 1), lambda i: (i, 0)),
            pl.BlockSpec((1, 2 * OUT), lambda i: (0, 0)),
            pl.BlockSpec((1, 2 * OUT), lambda i: (0, 0)),
            pl.BlockSpec((2 * OUT, OUT), lambda i: (0, 0)),
            pl.BlockSpec((1, OUT), lambda i: (0, 0)),
        ],
        out_specs=pl.BlockSpec((TN, OUT), lambda i: (i, 0)),
        out_shape=jax.ShapeDtypeStruct((N, OUT), jnp.float32),
    )(acc, degs2d, W0, b0_2d, W1, b1_2d)


def kernel(nfeat, efeat, n2p_idx, n2p_val, e2p_idx, e2p_val, pool_idx,
           pool_val, degs, weights, bias, W0, b0, W1, b1):
    mpad = M2 - n2p_idx.shape[0]
    ppad = P2 - pool_idx.shape[0]
    idxn = jnp.concatenate([n2p_idx.astype(jnp.int32),
                            jnp.zeros((mpad,), jnp.int32)])
    idxe = jnp.concatenate([e2p_idx.astype(jnp.int32),
                            jnp.zeros((mpad,), jnp.int32)])
    valn = jnp.concatenate([n2p_val, jnp.zeros((mpad,), jnp.float32)])
    vale = jnp.concatenate([e2p_val, jnp.zeros((mpad,), jnp.float32)])
    pidx = jnp.concatenate([pool_idx.astype(jnp.int32),
                            jnp.zeros((ppad,), jnp.int32)])
    pval = jnp.concatenate([pool_val, jnp.zeros((ppad,), jnp.float32)])

    y = _sc_gather(nfeat, efeat, idxn, idxe, valn, vale)
    x_flat = y.reshape(P2, L * IN)
    w_flat = weights.transpose(2, 0, 1).reshape(L * IN, OUT)
    z = _tc_matmul(x_flat, w_flat, bias, pval.reshape(P2, 1))
    acc = _sc_pool(z, pidx.reshape(NS, PCH * NSUB, SUB))
    out = _tc_final(acc, degs.reshape(N, 1), W0, b0.reshape(1, 2 * OUT),
                    W1, b1.reshape(1, OUT))
    return out
